# Initial kernel scaffold; baseline (speedup 1.0000x reference)
import jax
import jax.numpy as jnp
from jax.experimental import pallas as pl


def kernel(x_target, A_original, conv1_w, conv1_b, conv2_w, conv2_b, pair_w1, pair_b1, pair_w2, pair_b2, logit_w, logit_b, score_w, score_b, unk_w1, unk_b1, unk_w2, unk_b2):
    raise NotImplementedError("write your pallas kernel here")


# speedup vs baseline: 4.9793x; 4.9793x over previous
import functools

import jax
import jax.numpy as jnp
from jax.experimental import pallas as pl

B, N, T, C = 2, 325, 288, 1
H, RD, R = 64, 64, 8
TOPK, RHO, TEMP = 20, 0.5, 1.0

NP = 384
RB = 16
NRB = -(-N // RB)
NPR = NRB * RB
S = 16
BN = B * N
BNP = -(-BN // S) * S
E = N * (N - 1)


def _encoder_body(x_ref, w1m_ref, b1c_ref, m_ref, b2c_ref, w1a_ref, w1b_ref,
                  p_ref, q_ref):
    x = x_ref[...]
    z1 = jnp.zeros((x.shape[0], 1), jnp.float32)
    xl = jnp.concatenate([z1, x[:, :-1]], axis=1)
    xr = jnp.concatenate([x[:, 1:], z1], axis=1)
    w0 = w1m_ref[0:1, :].reshape(1, 1, H)
    w1 = w1m_ref[1:2, :].reshape(1, 1, H)
    w2 = w1m_ref[2:3, :].reshape(1, 1, H)
    h1 = (xl[:, :, None] * w0 + x[:, :, None] * w1 + xr[:, :, None] * w2
          + b1c_ref[0:1, :].reshape(1, 1, H))
    h1 = jnp.maximum(h1, 0.0)
    z1h = jnp.zeros((h1.shape[0], 1, H), jnp.float32)
    h1l = jnp.concatenate([z1h, h1[:, :-1, :]], axis=1)
    h1r = jnp.concatenate([h1[:, 1:, :], z1h], axis=1)
    hf = h1.reshape(-1, H)
    y = (jnp.dot(h1l.reshape(-1, H), m_ref[0], preferred_element_type=jnp.float32)
         + jnp.dot(hf, m_ref[1], preferred_element_type=jnp.float32)
         + jnp.dot(h1r.reshape(-1, H), m_ref[2], preferred_element_type=jnp.float32)
         + b2c_ref[0:1, :])
    h2 = jnp.maximum(y, 0.0)
    z = h2.reshape(-1, T, H).sum(axis=1) * (1.0 / T)
    p_ref[...] = jnp.dot(z, w1a_ref[...], preferred_element_type=jnp.float32)
    q_ref[...] = jnp.dot(z, w1b_ref[...], preferred_element_type=jnp.float32)


def _pair_body(p_ref, q_ref, ao_ref, b1_ref, w2_ref, b2_ref, wh_ref, bh_ref,
               uw1_ref, ub1_ref, uw2_ref, a_ref, acc_ref):
    bi = pl.program_id(0)
    rbi = pl.program_id(1)

    @pl.when(jnp.logical_and(bi == 0, rbi == 0))
    def _init():
        acc_ref[...] = jnp.zeros_like(acc_ref)

    p = p_ref[0]
    q = q_ref[0]
    h1 = jnp.maximum(p[:, None, :] + q[None, :, :]
                     + b1_ref[0:1, :].reshape(1, 1, RD), 0.0)
    hf = h1.reshape(RB * NP, RD)
    h2 = jnp.maximum(jnp.dot(hf, w2_ref[...], preferred_element_type=jnp.float32)
                     + b2_ref[0:1, :], 0.0)
    ls = jnp.dot(h2, wh_ref[...], preferred_element_type=jnp.float32) + bh_ref[0:1, :]
    logits = ls[:, :R]
    scores = ls[:, R:]
    lm = jnp.max(logits, axis=1, keepdims=True)
    ex = jnp.exp(logits - lm)
    attr = ex / jnp.sum(ex, axis=1, keepdims=True)
    rel = jax.nn.sigmoid(scores)
    known = jnp.sum(attr * rel, axis=1, keepdims=True)
    ent = -jnp.sum(attr * jnp.log(attr + 1e-12), axis=1, keepdims=True)
    knw = jnp.clip(1.0 - ent * (1.0 / jnp.log(float(R))), 0.0, 1.0)
    u = jnp.maximum(jnp.dot(h2, uw1_ref[...], preferred_element_type=jnp.float32)
                    + ub1_ref[0:1, :], 0.0)
    uwv = jnp.sum(u * uw2_ref[0:1, :], axis=1, keepdims=True)
    unknown = jax.nn.sigmoid(uwv)
    ewf = knw * known + (1.0 - knw) * unknown

    fi = jax.lax.broadcasted_iota(jnp.int32, (RB * NP, 1), 0)
    jf = fi % NP
    igf = rbi * RB + fi // NP
    okf = jnp.logical_and(jnp.logical_and(jf < N, igf < N), jf != igf)
    ent_sum = jnp.sum(jnp.where(okf, ent, 0.0), axis=0, keepdims=True)
    attr_sum = jnp.sum(jnp.where(okf, attr, 0.0), axis=0, keepdims=True)

    ew = ewf.reshape(RB, NP)
    ji = jax.lax.broadcasted_iota(jnp.int32, (RB, NP), 1)
    ig = rbi * RB + jax.lax.broadcasted_iota(jnp.int32, (RB, NP), 0)
    ok = jnp.logical_and(jnp.logical_and(ji < N, ig < N), ji != ig)
    lg = jnp.where(jnp.logical_and(ok, ew != 0.0), ew, -1e9)
    rm = jnp.max(lg, axis=1, keepdims=True)
    e = jnp.exp(lg - rm)
    sm = e / jnp.sum(e, axis=1, keepdims=True)
    sm = sm / jnp.maximum(jnp.sum(sm, axis=1, keepdims=True), 1e-12)

    w = jnp.where(ok, sm, -1.0)
    keep = jnp.zeros((RB, NP), jnp.float32)
    for _ in range(TOPK):
        m = jnp.max(w, axis=1, keepdims=True)
        is_m = w == m
        first = jnp.min(jnp.where(is_m, ji, NP), axis=1, keepdims=True)
        sel = jnp.logical_and(is_m, ji == first)
        w = jnp.where(sel, -2.0, w)
        keep = jnp.where(sel, 1.0, keep)
    a_topk = jnp.where(jnp.logical_and(keep > 0.0, ok), sm, 0.0)
    a_sum = jnp.sum(jnp.where(ig < N, a_topk, 0.0)).reshape(1, 1)

    a_ref[0] = (1.0 - RHO) * ao_ref[0] + RHO * a_topk

    acc_ref[0:1, 0:1] += a_sum
    acc_ref[1:2, 0:1] += ent_sum
    acc_ref[2:3, 0:R] += attr_sum


@functools.partial(jax.jit, static_argnums=())
def kernel(x_target, A_original, conv1_w, conv1_b, conv2_w, conv2_b,
           pair_w1, pair_b1, pair_w2, pair_b2, logit_w, logit_b,
           score_w, score_b, unk_w1, unk_b1, unk_w2, unk_b2):
    f32 = jnp.float32
    x = x_target.reshape(BN, T).astype(f32)
    x = jnp.pad(x, ((0, BNP - BN), (0, 0)))
    w1m = jnp.transpose(conv1_w[:, 0, :], (1, 0))
    m = jnp.transpose(conv2_w, (2, 1, 0))
    b1c = conv1_b.reshape(1, H)
    b2c = conv2_b.reshape(1, H)
    w1a = pair_w1[:H]
    w1b = pair_w1[H:]

    grid1 = (BNP // S,)
    p_flat, q_flat = pl.pallas_call(
        _encoder_body,
        grid=grid1,
        in_specs=[
            pl.BlockSpec((S, T), lambda i: (i, 0)),
            pl.BlockSpec((3, H), lambda i: (0, 0)),
            pl.BlockSpec((1, H), lambda i: (0, 0)),
            pl.BlockSpec((3, H, H), lambda i: (0, 0, 0)),
            pl.BlockSpec((1, H), lambda i: (0, 0)),
            pl.BlockSpec((H, RD), lambda i: (0, 0)),
            pl.BlockSpec((H, RD), lambda i: (0, 0)),
        ],
        out_specs=[
            pl.BlockSpec((S, RD), lambda i: (i, 0)),
            pl.BlockSpec((S, RD), lambda i: (i, 0)),
        ],
        out_shape=[
            jax.ShapeDtypeStruct((BNP, RD), f32),
            jax.ShapeDtypeStruct((BNP, RD), f32),
        ],
    )(x, w1m, b1c, m, b2c, w1a, w1b)

    p = jnp.pad(p_flat[:BN].reshape(B, N, RD), ((0, 0), (0, NPR - N), (0, 0)))
    q = jnp.pad(q_flat[:BN].reshape(B, N, RD), ((0, 0), (0, NP - N), (0, 0)))
    ao = jnp.pad(A_original.astype(f32),
                 ((0, 0), (0, NPR - N), (0, NP - N)))
    wh = jnp.concatenate([logit_w, score_w], axis=1)
    bh = jnp.concatenate([logit_b, score_b]).reshape(1, 2 * R)

    grid2 = (B, NRB)
    a_pad, acc = pl.pallas_call(
        _pair_body,
        grid=grid2,
        in_specs=[
            pl.BlockSpec((1, RB, RD), lambda b, r: (b, r, 0)),
            pl.BlockSpec((1, NP, RD), lambda b, r: (b, 0, 0)),
            pl.BlockSpec((1, RB, NP), lambda b, r: (b, r, 0)),
            pl.BlockSpec((1, RD), lambda b, r: (0, 0)),
            pl.BlockSpec((RD, RD), lambda b, r: (0, 0)),
            pl.BlockSpec((1, RD), lambda b, r: (0, 0)),
            pl.BlockSpec((RD, 2 * R), lambda b, r: (0, 0)),
            pl.BlockSpec((1, 2 * R), lambda b, r: (0, 0)),
            pl.BlockSpec((RD, RD), lambda b, r: (0, 0)),
            pl.BlockSpec((1, RD), lambda b, r: (0, 0)),
            pl.BlockSpec((1, RD), lambda b, r: (0, 0)),
        ],
        out_specs=[
            pl.BlockSpec((1, RB, NP), lambda b, r: (b, r, 0)),
            pl.BlockSpec((8, 128), lambda b, r: (0, 0)),
        ],
        out_shape=[
            jax.ShapeDtypeStruct((B, NPR, NP), f32),
            jax.ShapeDtypeStruct((8, 128), f32),
        ],
    )(p, q, ao, pair_b1.reshape(1, RD), pair_w2, pair_b2.reshape(1, RD),
      wh, bh, unk_w1, unk_b1.reshape(1, RD), unk_w2.reshape(1, RD))

    a_final = a_pad[:, :N, :N]
    sparse_l = acc[0, 0] / (B * N * N)
    sharp_l = acc[1, 0] / (B * E)
    usage = acc[2, :R] / (B * E)
    balance_l = jnp.sum((usage - 1.0 / R) ** 2) * R
    return (a_final, sparse_l, sharp_l, balance_l)


# speedup vs baseline: 8.8750x; 1.7824x over previous
import functools

import jax
import jax.numpy as jnp
from jax.experimental import pallas as pl

B, N, T, C = 2, 325, 288, 1
H, RD, R = 64, 64, 8
TOPK, RHO, TEMP = 20, 0.5, 1.0

NP = 384
RB = 16
NRB = -(-N // RB)
NPR = NRB * RB
S = 16
BN = B * N
BNP = -(-BN // S) * S
E = N * (N - 1)


def _encoder_body(x_ref, w1m_ref, b1c_ref, m_ref, b2c_ref, w1a_ref, w1b_ref,
                  p_ref, q_ref):
    x = x_ref[...]
    z1 = jnp.zeros((x.shape[0], 1), jnp.float32)
    xl = jnp.concatenate([z1, x[:, :-1]], axis=1)
    xr = jnp.concatenate([x[:, 1:], z1], axis=1)
    w0 = w1m_ref[0:1, :].reshape(1, 1, H)
    w1 = w1m_ref[1:2, :].reshape(1, 1, H)
    w2 = w1m_ref[2:3, :].reshape(1, 1, H)
    h1 = (xl[:, :, None] * w0 + x[:, :, None] * w1 + xr[:, :, None] * w2
          + b1c_ref[0:1, :].reshape(1, 1, H))
    h1 = jnp.maximum(h1, 0.0)
    z1h = jnp.zeros((h1.shape[0], 1, H), jnp.float32)
    h1l = jnp.concatenate([z1h, h1[:, :-1, :]], axis=1)
    h1r = jnp.concatenate([h1[:, 1:, :], z1h], axis=1)
    hf = h1.reshape(-1, H)
    y = (jnp.dot(h1l.reshape(-1, H), m_ref[0], preferred_element_type=jnp.float32)
         + jnp.dot(hf, m_ref[1], preferred_element_type=jnp.float32)
         + jnp.dot(h1r.reshape(-1, H), m_ref[2], preferred_element_type=jnp.float32)
         + b2c_ref[0:1, :])
    h2 = jnp.maximum(y, 0.0)
    z = h2.reshape(-1, T, H).sum(axis=1) * (1.0 / T)
    p_ref[...] = jnp.dot(z, w1a_ref[...], preferred_element_type=jnp.float32)
    q_ref[...] = jnp.dot(z, w1b_ref[...], preferred_element_type=jnp.float32)


def _pair_body(pt_ref, qt_ref, ao_ref, b1t_ref, w2t_ref, b2t_ref, wht_ref,
               bht_ref, uw1t_ref, ub1t_ref, uw2r_ref, ub2_ref, a_ref, acc_ref):
    bi = pl.program_id(0)
    rbi = pl.program_id(1)

    @pl.when(jnp.logical_and(bi == 0, rbi == 0))
    def _init():
        acc_ref[...] = jnp.zeros_like(acc_ref)

    pT = pt_ref[0, 0]
    qT = qt_ref[0]
    b1T = b1t_ref[...]
    cols = [jnp.maximum(pT[:, i:i + 1] + qT + b1T, 0.0) for i in range(RB)]
    h1w = jnp.concatenate(cols, axis=1)
    h2w = jnp.maximum(
        jnp.dot(w2t_ref[...], h1w, preferred_element_type=jnp.float32)
        + b2t_ref[...], 0.0)
    lsw = (jnp.dot(wht_ref[...], h2w, preferred_element_type=jnp.float32)
           + bht_ref[...])
    logits = lsw[0:R]
    scores = lsw[R:2 * R]
    lm = jnp.max(logits, axis=0, keepdims=True)
    ex = jnp.exp(logits - lm)
    attr = ex / jnp.sum(ex, axis=0, keepdims=True)
    rel = jax.nn.sigmoid(scores)
    known = jnp.sum(attr * rel, axis=0, keepdims=True)
    ent = -jnp.sum(attr * jnp.log(attr + 1e-12), axis=0, keepdims=True)
    knw = jnp.clip(1.0 - ent * (1.0 / jnp.log(float(R))), 0.0, 1.0)
    u = jnp.maximum(
        jnp.dot(uw1t_ref[...], h2w, preferred_element_type=jnp.float32)
        + ub1t_ref[...], 0.0)
    uwv = (jnp.dot(uw2r_ref[...], u, preferred_element_type=jnp.float32)
           + ub2_ref[...])
    unknown = jax.nn.sigmoid(uwv)
    ewf = knw * known + (1.0 - knw) * unknown

    fi = jax.lax.broadcasted_iota(jnp.int32, (1, RB * NP), 1)
    jf = fi % NP
    igf = rbi * RB + fi // NP
    okf = jnp.logical_and(jnp.logical_and(jf < N, igf < N), jf != igf)
    ent_sum = jnp.sum(jnp.where(okf, ent, 0.0), axis=1, keepdims=True)
    attr_sum = jnp.sum(jnp.where(okf, attr, 0.0), axis=1, keepdims=True)

    ew = ewf.reshape(RB, NP)
    ji = jax.lax.broadcasted_iota(jnp.int32, (RB, NP), 1)
    ig = rbi * RB + jax.lax.broadcasted_iota(jnp.int32, (RB, NP), 0)
    ok = jnp.logical_and(jnp.logical_and(ji < N, ig < N), ji != ig)
    lg = jnp.where(jnp.logical_and(ok, ew != 0.0), ew, -1e9)
    rm = jnp.max(lg, axis=1, keepdims=True)
    e = jnp.exp(lg - rm)
    sm = e / jnp.sum(e, axis=1, keepdims=True)
    sm = sm / jnp.maximum(jnp.sum(sm, axis=1, keepdims=True), 1e-12)

    w = jnp.where(ok, sm, -1.0)
    keep = jnp.zeros((RB, NP), jnp.float32)
    for _ in range(TOPK):
        m = jnp.max(w, axis=1, keepdims=True)
        is_m = w == m
        first = jnp.min(jnp.where(is_m, ji, NP), axis=1, keepdims=True)
        sel = jnp.logical_and(is_m, ji == first)
        w = jnp.where(sel, -2.0, w)
        keep = jnp.where(sel, 1.0, keep)
    a_topk = jnp.where(jnp.logical_and(keep > 0.0, ok), sm, 0.0)
    a_sum = jnp.sum(jnp.where(ig < N, a_topk, 0.0)).reshape(1, 1)

    a_ref[0] = (1.0 - RHO) * ao_ref[0] + RHO * a_topk

    acc_ref[0:1, 0:1] += a_sum
    acc_ref[1:2, 0:1] += ent_sum
    acc_ref[2:2 + R, 0:1] += attr_sum


@functools.partial(jax.jit, static_argnums=())
def kernel(x_target, A_original, conv1_w, conv1_b, conv2_w, conv2_b,
           pair_w1, pair_b1, pair_w2, pair_b2, logit_w, logit_b,
           score_w, score_b, unk_w1, unk_b1, unk_w2, unk_b2):
    f32 = jnp.float32
    x = x_target.reshape(BN, T).astype(f32)
    x = jnp.pad(x, ((0, BNP - BN), (0, 0)))
    w1m = jnp.transpose(conv1_w[:, 0, :], (1, 0))
    m = jnp.transpose(conv2_w, (2, 1, 0))
    b1c = conv1_b.reshape(1, H)
    b2c = conv2_b.reshape(1, H)
    w1a = pair_w1[:H]
    w1b = pair_w1[H:]

    grid1 = (BNP // S,)
    p_flat, q_flat = pl.pallas_call(
        _encoder_body,
        grid=grid1,
        in_specs=[
            pl.BlockSpec((S, T), lambda i: (i, 0)),
            pl.BlockSpec((3, H), lambda i: (0, 0)),
            pl.BlockSpec((1, H), lambda i: (0, 0)),
            pl.BlockSpec((3, H, H), lambda i: (0, 0, 0)),
            pl.BlockSpec((1, H), lambda i: (0, 0)),
            pl.BlockSpec((H, RD), lambda i: (0, 0)),
            pl.BlockSpec((H, RD), lambda i: (0, 0)),
        ],
        out_specs=[
            pl.BlockSpec((S, RD), lambda i: (i, 0)),
            pl.BlockSpec((S, RD), lambda i: (i, 0)),
        ],
        out_shape=[
            jax.ShapeDtypeStruct((BNP, RD), f32),
            jax.ShapeDtypeStruct((BNP, RD), f32),
        ],
    )(x, w1m, b1c, m, b2c, w1a, w1b)

    pt = jnp.transpose(
        jnp.pad(p_flat[:BN].reshape(B, N, RD), ((0, 0), (0, NPR - N), (0, 0)))
        .reshape(B, NRB, RB, RD),
        (0, 1, 3, 2))
    qt = jnp.transpose(
        jnp.pad(q_flat[:BN].reshape(B, N, RD), ((0, 0), (0, NP - N), (0, 0))),
        (0, 2, 1))
    ao = jnp.pad(A_original.astype(f32),
                 ((0, 0), (0, NPR - N), (0, NP - N)))
    wht = jnp.concatenate([logit_w, score_w], axis=1).T
    bht = jnp.concatenate([logit_b, score_b]).reshape(2 * R, 1)

    grid2 = (B, NRB)
    a_pad, acc = pl.pallas_call(
        _pair_body,
        grid=grid2,
        in_specs=[
            pl.BlockSpec((1, 1, RD, RB), lambda b, r: (b, r, 0, 0)),
            pl.BlockSpec((1, RD, NP), lambda b, r: (b, 0, 0)),
            pl.BlockSpec((1, RB, NP), lambda b, r: (b, r, 0)),
            pl.BlockSpec((RD, 1), lambda b, r: (0, 0)),
            pl.BlockSpec((RD, RD), lambda b, r: (0, 0)),
            pl.BlockSpec((RD, 1), lambda b, r: (0, 0)),
            pl.BlockSpec((2 * R, RD), lambda b, r: (0, 0)),
            pl.BlockSpec((2 * R, 1), lambda b, r: (0, 0)),
            pl.BlockSpec((RD, RD), lambda b, r: (0, 0)),
            pl.BlockSpec((RD, 1), lambda b, r: (0, 0)),
            pl.BlockSpec((1, RD), lambda b, r: (0, 0)),
            pl.BlockSpec((1, 1), lambda b, r: (0, 0)),
        ],
        out_specs=[
            pl.BlockSpec((1, RB, NP), lambda b, r: (b, r, 0)),
            pl.BlockSpec((16, 128), lambda b, r: (0, 0)),
        ],
        out_shape=[
            jax.ShapeDtypeStruct((B, NPR, NP), f32),
            jax.ShapeDtypeStruct((16, 128), f32),
        ],
    )(pt, qt, ao, pair_b1.reshape(RD, 1), pair_w2.T, pair_b2.reshape(RD, 1),
      wht, bht, unk_w1.T, unk_b1.reshape(RD, 1), unk_w2.reshape(1, RD),
      unk_b2.reshape(1, 1))

    a_final = a_pad[:, :N, :N]
    sparse_l = acc[0, 0] / (B * N * N)
    sharp_l = acc[1, 0] / (B * E)
    usage = acc[2:2 + R, 0] / (B * E)
    balance_l = jnp.sum((usage - 1.0 / R) ** 2) * R
    return (a_final, sparse_l, sharp_l, balance_l)
